# baseline (speedup 1.0000x reference)
import functools

import jax
import jax.numpy as jnp
from jax import lax
from jax.experimental import pallas as pl
from jax.experimental.pallas import tpu as pltpu
from jax.experimental.pallas import tpu_sc as plsc

N = 50000
E = 800000
NELEM = 89
MAXN = 9
D = 64
CUTOFF = 5.0
NBLK = 4
H = 32

NTILE = 16
NPAD = 50688
EPAD = 804864
CH = 96
NCHUNK = EPAD // NTILE // CH
EPT = EPAD // NTILE
ROWS_PER_TILE = NPAD // NTILE
NXCH = ROWS_PER_TILE // CH

RMIN = 0.5
BINS = 16384
SCALE = BINS / (CUTOFF - RMIN)

_MESH = plsc.VectorSubcoreMesh(core_axis_name="c", subcore_axis_name="s")
_SC_PARAMS = pltpu.CompilerParams(use_tc_tiling_on_sc=False,
                                  needs_layout_passes=False)

_HIMASK = -65536


def _lo(xi):
    return plsc.bitcast(lax.shift_left(xi, 16), jnp.float32)


def _hi(xi):
    return plsc.bitcast(lax.bitwise_and(xi, _HIMASK), jnp.float32)


def _emb_body(emb_hbm, nidx_hbm, x_hbm, idxb, rows, sem):
    c = lax.axis_index("c")
    s = lax.axis_index("s")
    base = s * ROWS_PER_TILE

    @pl.loop(0, NXCH)
    def _chunk(g):
        off = base + g * CH
        pltpu.sync_copy(nidx_hbm.at[c, pl.ds(off, CH)], idxb)
        pltpu.async_copy(emb_hbm.at[idxb], rows, sem).wait()
        pltpu.sync_copy(rows, x_hbm.at[c, pl.ds(off, CH), :])


_emb_kernel = functools.partial(
    pl.kernel,
    out_type=jax.ShapeDtypeStruct((2, NPAD, H), jnp.float32),
    mesh=_MESH,
    scratch_types=[
        pltpu.VMEM((CH,), jnp.int32),
        pltpu.VMEM((CH, H), jnp.float32),
        pltpu.SemaphoreType.DMA,
    ],
    compiler_params=_SC_PARAMS,
)(_emb_body)


_EB = 1024


def _qidx_body(bd_ref, q_ref):
    r = bd_ref[...]
    q = jnp.clip((r - RMIN) * SCALE, 0.0, BINS - 1.0).astype(jnp.int32)
    offs = (lax.broadcasted_iota(jnp.int32, (NBLK, 2, 1), 0) * 2
            + lax.broadcasted_iota(jnp.int32, (NBLK, 2, 1), 1)) * BINS
    q_ref[...] = jnp.broadcast_to(q.reshape(1, 1, _EB),
                                  (NBLK, 2, _EB)) + offs


def _qidx_kernel(bd1):
    return pl.pallas_call(
        _qidx_body,
        grid=(EPAD // _EB,),
        in_specs=[pl.BlockSpec((1, _EB), lambda e: (0, e))],
        out_specs=pl.BlockSpec((NBLK, 2, _EB), lambda e: (0, 0, e)),
        out_shape=jax.ShapeDtypeStruct((NBLK, 2, EPAD), jnp.int32),
    )(bd1)


_TB = 1024


def _table_body(bw_ref, awp_ref, wcg_ref, bcg_ref, cw_ref):
    g = pl.program_id(0)
    b = (lax.broadcasted_iota(jnp.int32, (_TB, 1), 0)
         + g * _TB).astype(jnp.float32)
    r = RMIN + (b + 0.5) * (1.0 / SCALE)
    x = r * (1.0 / CUTOFF)
    x2 = x * x
    x5 = x2 * x2 * x
    f = 1.0 - 21.0 * x5 + 35.0 * x5 * x - 15.0 * x5 * x2
    fc = jnp.where(x < 1.0, f, 0.0)
    freqs = ((lax.broadcasted_iota(jnp.int32, (1, MAXN), 1) + 1)
             .astype(jnp.float32) * (jnp.pi / CUTOFF))
    rbf = jnp.sqrt(2.0 / CUTOFF) * jnp.sin(r * freqs) / r
    bf = jnp.dot(rbf, bw_ref[...], preferred_element_type=jnp.float32)
    abw = jnp.dot(rbf, awp_ref[...], preferred_element_type=jnp.float32)
    w = (abw * fc).astype(jnp.bfloat16)
    for i in range(NBLK):
        t = (jnp.dot(bf, wcg_ref[i], preferred_element_type=jnp.float32)
             + bcg_ref[i]).astype(jnp.bfloat16)
        cw_ref[i, 0] = jnp.concatenate([t[:, :D], w[:, :H]], axis=1)
        cw_ref[i, 1] = jnp.concatenate([t[:, D:], w[:, H:]], axis=1)


def _table_kernel(bond_W, awp, WCG, bcg):
    full = lambda shape: pl.BlockSpec(shape, lambda g: tuple(0 for _ in shape))
    return pl.pallas_call(
        _table_body,
        grid=(BINS // _TB,),
        in_specs=[
            full((MAXN, D)),
            full((MAXN, D)),
            full((NBLK, D, 2 * D)),
            full((NBLK, 1, 2 * D)),
        ],
        out_specs=pl.BlockSpec((NBLK, 2, _TB, 96), lambda g: (0, 0, g, 0)),
        out_shape=jax.ShapeDtypeStruct((NBLK, 2, BINS, 96), jnp.bfloat16),
    )(bond_W, awp, WCG, bcg)


_NB = 512


def _ntrans_body(x_ref, m_ref, t_ref):
    z = jnp.concatenate([x_ref[0], x_ref[1]], axis=-1)
    t_ref[...] = jnp.dot(
        z, m_ref[0], preferred_element_type=jnp.float32
    ).astype(jnp.bfloat16)


def _node_transform(X, M):
    return pl.pallas_call(
        _ntrans_body,
        grid=(4, NPAD // _NB),
        in_specs=[
            pl.BlockSpec((2, _NB, H), lambda t, n: (0, n, 0)),
            pl.BlockSpec((1, D, D), lambda t, n: (t, 0, 0)),
        ],
        out_specs=pl.BlockSpec((_NB, D),
                               lambda t, n: (t * (NPAD // _NB) + n, 0)),
        out_shape=jax.ShapeDtypeStruct((4 * NPAD, D), jnp.bfloat16),
    )(X, M)


def _make_edge_body(blk):
    def body(t_hbm, uidx_hbm, vidx_hbm, dst_hbm, qidx_hbm, cw_hbm,
             xin_hbm, xout_hbm,
             acc, ubuf, vbuf, cwbuf, msgb, uix, vix, cix, dix,
             semL0, semL1, semI0, semI1, semS0, semS1, semD0, semD1):
        c = lax.axis_index("c")
        s = lax.axis_index("s")
        rbase = s * ROWS_PER_TILE
        ebase = s * EPT
        semL = (semL0, semL1)
        semI = (semI0, semI1)
        semS = (semS0, semS1)
        semD = (semD0, semD1)

        @pl.loop(0, NXCH)
        def _init(j):
            off = rbase + j * CH
            pltpu.sync_copy(xin_hbm.at[c, pl.ds(off, CH), :], msgb.at[0])
            pltpu.sync_copy(msgb.at[0], acc.at[pl.ds(off, CH), :])

        plsc.subcore_barrier()

        def eoff(ci):
            return ebase + lax.rem(ci, NCHUNK) * CH

        def issue_idx(ci, slot, sem):
            o = eoff(ci)
            pltpu.async_copy(uidx_hbm.at[c, pl.ds(o, CH)], uix.at[slot], sem)
            pltpu.async_copy(vidx_hbm.at[c, pl.ds(o, CH)], vix.at[slot], sem)
            pltpu.async_copy(qidx_hbm.at[blk, c, pl.ds(o, CH)],
                             cix.at[slot], sem)

        def wait_idx(ci, slot, sem):
            o = eoff(ci)
            pltpu.make_async_copy(
                uidx_hbm.at[c, pl.ds(o, CH)], uix.at[slot], sem).wait()
            pltpu.make_async_copy(
                vidx_hbm.at[c, pl.ds(o, CH)], vix.at[slot], sem).wait()
            pltpu.make_async_copy(
                qidx_hbm.at[blk, c, pl.ds(o, CH)], cix.at[slot], sem).wait()

        def issue_loads(b, slot, sem):
            pltpu.async_copy(cw_hbm.at[cix.at[slot]], cwbuf.at[b], sem)
            pltpu.async_copy(t_hbm.at[uix.at[slot]], ubuf.at[b], sem)
            pltpu.async_copy(t_hbm.at[vix.at[slot]], vbuf.at[b], sem)

        def wait_loads(b, slot, sem):
            pltpu.make_async_copy(cw_hbm.at[cix.at[slot]], cwbuf.at[b],
                                  sem).wait()
            pltpu.make_async_copy(t_hbm.at[uix.at[slot]], ubuf.at[b],
                                  sem).wait()
            pltpu.make_async_copy(t_hbm.at[vix.at[slot]], vbuf.at[b],
                                  sem).wait()

        pltpu.sync_copy(uidx_hbm.at[c, pl.ds(ebase, CH)], uix.at[0])
        pltpu.sync_copy(vidx_hbm.at[c, pl.ds(ebase, CH)], vix.at[0])
        pltpu.sync_copy(qidx_hbm.at[blk, c, pl.ds(ebase, CH)], cix.at[0])
        pltpu.sync_copy(uidx_hbm.at[c, pl.ds(ebase + CH, CH)], uix.at[1])
        pltpu.sync_copy(vidx_hbm.at[c, pl.ds(ebase + CH, CH)], vix.at[1])
        pltpu.sync_copy(qidx_hbm.at[blk, c, pl.ds(ebase + CH, CH)], cix.at[1])
        issue_idx(2, 2, semI[0])
        issue_idx(3, 3, semI[1])
        issue_loads(0, 0, semL[0])
        issue_loads(1, 1, semL[1])
        zeros16 = jnp.zeros((16,), jnp.float32)
        trash = jnp.full((16,), NPAD, jnp.int32)
        for b in range(2):
            @pl.loop(0, CH)
            def _z(r):
                msgb[b, r, pl.ds(0, 16)] = zeros16
                msgb[b, r, pl.ds(16, 16)] = zeros16

            for j in range(CH // 16):
                dix[b, pl.ds(j * 16, 16)] = trash
            pltpu.async_copy(msgb.at[b], acc.at[dix.at[b]], semS[b], add=True)

        @pl.loop(0, NCHUNK // 4)
        def _quad(g):
            for u in range(4):
                b = u % 2
                ci = g * 4 + u
                pltpu.make_async_copy(msgb.at[b], acc.at[dix.at[b]],
                                      semS[b]).wait()
                pltpu.async_copy(dst_hbm.at[pl.ds(eoff(ci), CH)],
                                 dix.at[b], semD[b])
                wait_loads(b, u, semL[b])

                @plsc.parallel_loop(0, CH, unroll=4)
                def _edge(ii):
                    wi = plsc.bitcast(cwbuf[b, ii, pl.ds(2 * H, H)], jnp.int32)
                    for k in range(2):
                        sl32 = pl.ds(k * H, H)
                        ui = plsc.bitcast(ubuf[b, ii, sl32], jnp.int32)
                        vi = plsc.bitcast(vbuf[b, ii, sl32], jnp.int32)
                        ci_ = plsc.bitcast(cwbuf[b, ii, sl32], jnp.int32)
                        cp = _lo(ui) + _lo(vi) + _lo(ci_)
                        gp = _hi(ui) + _hi(vi) + _hi(ci_)
                        w = _lo(wi) if k == 0 else _hi(wi)
                        den = (1.0 + jnp.exp(-cp)) * (1.0 + jnp.exp(-gp))
                        msgb[b, ii, pl.ds(k * 16, 16)] = cp * w / den

                pltpu.make_async_copy(dst_hbm.at[pl.ds(eoff(ci), CH)],
                                      dix.at[b], semD[b]).wait()
                pltpu.async_copy(msgb.at[b], acc.at[dix.at[b]],
                                 semS[b], add=True)
                wait_idx(ci + 2, (u + 2) % 4, semI[b])
                issue_loads(b, (u + 2) % 4, semL[b])
                issue_idx(ci + 4, u, semI[b])

        for u in range(2):
            b = u % 2
            pltpu.make_async_copy(msgb.at[b], acc.at[dix.at[b]],
                                  semS[b]).wait()
            wait_loads(b, (u + 2) % 4, semL[b])
            wait_idx(NCHUNK + u + 2, u, semI[b])

        plsc.subcore_barrier()

        @pl.loop(0, NXCH)
        def _out(j):
            off = rbase + j * CH
            pltpu.sync_copy(acc.at[pl.ds(off, CH), :], msgb.at[0])
            pltpu.sync_copy(msgb.at[0], xout_hbm.at[c, pl.ds(off, CH), :])

    return body


def _edge_kernel(blk):
    return pl.kernel(
        _make_edge_body(blk),
        out_type=jax.ShapeDtypeStruct((2, NPAD, H), jnp.float32),
        mesh=_MESH,
        scratch_types=[
            pltpu.VMEM_SHARED((NPAD + 8, H), jnp.float32),
            pltpu.VMEM((2, CH, D), jnp.bfloat16),
            pltpu.VMEM((2, CH, D), jnp.bfloat16),
            pltpu.VMEM((2, CH, 96), jnp.bfloat16),
            pltpu.VMEM((2, CH, H), jnp.float32),
            pltpu.VMEM((4, CH), jnp.int32),
            pltpu.VMEM((4, CH), jnp.int32),
            pltpu.VMEM((4, CH), jnp.int32),
            pltpu.VMEM((2, CH), jnp.int32),
            pltpu.SemaphoreType.DMA,
            pltpu.SemaphoreType.DMA,
            pltpu.SemaphoreType.DMA,
            pltpu.SemaphoreType.DMA,
            pltpu.SemaphoreType.DMA,
            pltpu.SemaphoreType.DMA,
            pltpu.SemaphoreType.DMA,
            pltpu.SemaphoreType.DMA,
        ],
        compiler_params=_SC_PARAMS,
    )


_RB = 1000


def _readout_body(x_ref, r1w, r1b, r2w, r2b, r3w, r3b, sw, sb,
                  site_ref, en_ref):
    z = jnp.concatenate([x_ref[0], x_ref[1]], axis=-1)
    site_ref[...] = jnp.dot(z, sw[...],
                            preferred_element_type=jnp.float32) + sb[...]
    h = jnp.dot(z, r1w[...], preferred_element_type=jnp.float32) + r1b[...]
    h = h / (1.0 + jnp.exp(-h))
    h = jnp.dot(h, r2w[...], preferred_element_type=jnp.float32) + r2b[...]
    h = h / (1.0 + jnp.exp(-h))
    pn = jnp.dot(h, r3w[...], preferred_element_type=jnp.float32) + r3b[...]
    en = jnp.sum(pn)

    @pl.when(pl.program_id(0) == 0)
    def _():
        en_ref[...] = jnp.zeros((1, 1), jnp.float32)

    en_ref[...] += jnp.reshape(en, (1, 1))


def _readout(X, R1W, R1b, R2W, R2b, R3W, R3b, siteW, siteb):
    full = lambda shape: pl.BlockSpec(shape, lambda n: tuple(0 for _ in shape))
    return pl.pallas_call(
        _readout_body,
        grid=(N // _RB,),
        in_specs=[
            pl.BlockSpec((2, _RB, H), lambda n: (0, n, 0)),
            full((D, D)), full((1, D)),
            full((D, D)), full((1, D)),
            full((D, 1)), full((1, 1)),
            full((D, 1)), full((1, 1)),
        ],
        out_specs=[
            pl.BlockSpec((_RB, 1), lambda n: (n, 0)),
            pl.BlockSpec((1, 1), lambda n: (0, 0)),
        ],
        out_shape=[
            jax.ShapeDtypeStruct((N, 1), jnp.float32),
            jax.ShapeDtypeStruct((1, 1), jnp.float32),
        ],
    )(X, R1W, R1b, R2W, R2b, R3W, R3b, siteW, siteb)


def kernel(node_types, edge_index, bond_dist, atom_emb, bond_W, abw_W,
           Wc, bc, Wg, bg, R1W, R1b, R2W, R2b, R3W, R3b, siteW, siteb):
    f32 = jnp.float32
    src = edge_index[0].astype(jnp.int32)
    dst = edge_index[1].astype(jnp.int32)
    nt = node_types.astype(jnp.int32)

    src_p = jnp.pad(src, (0, EPAD - E))
    dst_p = jnp.pad(dst, (0, EPAD - E))
    bd_p = jnp.pad(bond_dist.astype(f32), (0, EPAD - E),
                   constant_values=2.0 * CUTOFF)
    nt_p = jnp.pad(nt, (0, NPAD - N))

    uidx = jnp.stack([src_p, src_p + NPAD])
    vidx = jnp.stack([dst_p + 2 * NPAD, dst_p + 3 * NPAD])
    nidx = jnp.stack([nt_p, nt_p + NELEM])

    embS = jnp.concatenate([atom_emb[:, :H], atom_emb[:, H:]], axis=0)
    embS = embS.astype(f32)
    wcc = Wc[:, 2 * D:, :].reshape(NBLK, D, 2, 2, 16)
    wgc = Wg[:, 2 * D:, :].reshape(NBLK, D, 2, 2, 16)
    WCG = jnp.stack([wcc, wgc], axis=-1).reshape(NBLK, D, 2 * D).astype(f32)
    bcr = bc.reshape(NBLK, 2, 2, 16)
    bgr = bg.reshape(NBLK, 2, 2, 16)
    bcg = jnp.stack([bcr, bgr], axis=-1).reshape(NBLK, 1, 2 * D).astype(f32)
    awp = abw_W.reshape(MAXN, 2, 2, 16).transpose(0, 1, 3, 2)
    awp = awp.reshape(MAXN, D).astype(f32)
    Ms = []
    for i in range(NBLK):
        wa_c = Wc[i, :D, :].reshape(D, 2, 2, 16)
        wb_c = Wc[i, D:2 * D, :].reshape(D, 2, 2, 16)
        wa_g = Wg[i, :D, :].reshape(D, 2, 2, 16)
        wb_g = Wg[i, D:2 * D, :].reshape(D, 2, 2, 16)
        tbls = [
            jnp.stack([wa_c[:, 0], wa_g[:, 0]], axis=-1).reshape(D, D),
            jnp.stack([wa_c[:, 1], wa_g[:, 1]], axis=-1).reshape(D, D),
            jnp.stack([wb_c[:, 0], wb_g[:, 0]], axis=-1).reshape(D, D),
            jnp.stack([wb_c[:, 1], wb_g[:, 1]], axis=-1).reshape(D, D),
        ]
        Ms.append(jnp.stack(tbls).astype(f32))

    QIDX = _qidx_kernel(bd_p.reshape(1, EPAD))
    CW = _table_kernel(bond_W.astype(f32), awp, WCG, bcg)
    CWflat = CW.reshape(NBLK * 2 * BINS, 96)

    X = _emb_kernel(embS, nidx)

    for i in range(NBLK):
        Tflat = _node_transform(X, Ms[i])
        X = _edge_kernel(i)(Tflat, uidx, vidx, dst_p, QIDX, CWflat, X)

    site, en = _readout(X, R1W.astype(f32), R1b.reshape(1, D).astype(f32),
                        R2W.astype(f32), R2b.reshape(1, D).astype(f32),
                        R3W.astype(f32), R3b.reshape(1, 1).astype(f32),
                        siteW.astype(f32), siteb.reshape(1, 1).astype(f32))
    return (en.reshape(1), site)


# speedup vs baseline: 1.1796x; 1.1796x over previous
import functools

import jax
import jax.numpy as jnp
from jax import lax
from jax.experimental import pallas as pl
from jax.experimental.pallas import tpu as pltpu
from jax.experimental.pallas import tpu_sc as plsc

N = 50000
E = 800000
NELEM = 89
MAXN = 9
D = 64
CUTOFF = 5.0
NBLK = 4
H = 32

NTILE = 16
NPAD = 50688
EPAD = 804864
CH = 96
NCHUNK = EPAD // NTILE // CH
EPT = EPAD // NTILE
ROWS_PER_TILE = NPAD // NTILE
NXCH = ROWS_PER_TILE // CH

RMIN = 0.5
BINS = 16384
SCALE = BINS / (CUTOFF - RMIN)

_MESH = plsc.VectorSubcoreMesh(core_axis_name="c", subcore_axis_name="s")
_SC_PARAMS = pltpu.CompilerParams(use_tc_tiling_on_sc=False,
                                  needs_layout_passes=False)

_HIMASK = -65536


def _lo(xi):
    return plsc.bitcast(lax.shift_left(xi, 16), jnp.float32)


def _hi(xi):
    return plsc.bitcast(lax.bitwise_and(xi, _HIMASK), jnp.float32)


def _emb_body(emb_hbm, nidx_hbm, x_hbm, idxb, rows, sem):
    c = lax.axis_index("c")
    s = lax.axis_index("s")
    base = s * ROWS_PER_TILE

    @pl.loop(0, NXCH)
    def _chunk(g):
        off = base + g * CH
        pltpu.sync_copy(nidx_hbm.at[c, pl.ds(off, CH)], idxb)
        pltpu.async_copy(emb_hbm.at[idxb], rows, sem).wait()
        pltpu.sync_copy(rows, x_hbm.at[c, pl.ds(off, CH), :])


_emb_kernel = functools.partial(
    pl.kernel,
    out_type=jax.ShapeDtypeStruct((2, NPAD, H), jnp.float32),
    mesh=_MESH,
    scratch_types=[
        pltpu.VMEM((CH,), jnp.int32),
        pltpu.VMEM((CH, H), jnp.float32),
        pltpu.SemaphoreType.DMA,
    ],
    compiler_params=_SC_PARAMS,
)(_emb_body)


_EB = 1024


def _qidx_body(bd_ref, q_ref):
    r = bd_ref[...]
    q = jnp.clip((r - RMIN) * SCALE, 0.0, BINS - 1.0).astype(jnp.int32)
    offs = (lax.broadcasted_iota(jnp.int32, (NBLK, 2, 1), 0) * 2
            + lax.broadcasted_iota(jnp.int32, (NBLK, 2, 1), 1)) * BINS
    q_ref[...] = jnp.broadcast_to(q.reshape(1, 1, _EB),
                                  (NBLK, 2, _EB)) + offs


def _qidx_kernel(bd1):
    return pl.pallas_call(
        _qidx_body,
        grid=(EPAD // _EB,),
        in_specs=[pl.BlockSpec((1, _EB), lambda e: (0, e))],
        out_specs=pl.BlockSpec((NBLK, 2, _EB), lambda e: (0, 0, e)),
        out_shape=jax.ShapeDtypeStruct((NBLK, 2, EPAD), jnp.int32),
    )(bd1)


_TB = 1024


def _table_body(bw_ref, awp_ref, wcg_ref, bcg_ref, cw_ref):
    g = pl.program_id(0)
    b = (lax.broadcasted_iota(jnp.int32, (_TB, 1), 0)
         + g * _TB).astype(jnp.float32)
    r = RMIN + (b + 0.5) * (1.0 / SCALE)
    x = r * (1.0 / CUTOFF)
    x2 = x * x
    x5 = x2 * x2 * x
    f = 1.0 - 21.0 * x5 + 35.0 * x5 * x - 15.0 * x5 * x2
    fc = jnp.where(x < 1.0, f, 0.0)
    freqs = ((lax.broadcasted_iota(jnp.int32, (1, MAXN), 1) + 1)
             .astype(jnp.float32) * (jnp.pi / CUTOFF))
    rbf = jnp.sqrt(2.0 / CUTOFF) * jnp.sin(r * freqs) / r
    bf = jnp.dot(rbf, bw_ref[...], preferred_element_type=jnp.float32)
    abw = jnp.dot(rbf, awp_ref[...], preferred_element_type=jnp.float32)
    w = (abw * fc).astype(jnp.bfloat16)
    for i in range(NBLK):
        t = (jnp.dot(bf, wcg_ref[i], preferred_element_type=jnp.float32)
             + bcg_ref[i]).astype(jnp.bfloat16)
        cw_ref[i, 0] = jnp.concatenate([t[:, :D], w[:, :H]], axis=1)
        cw_ref[i, 1] = jnp.concatenate([t[:, D:], w[:, H:]], axis=1)


def _table_kernel(bond_W, awp, WCG, bcg):
    full = lambda shape: pl.BlockSpec(shape, lambda g: tuple(0 for _ in shape))
    return pl.pallas_call(
        _table_body,
        grid=(BINS // _TB,),
        in_specs=[
            full((MAXN, D)),
            full((MAXN, D)),
            full((NBLK, D, 2 * D)),
            full((NBLK, 1, 2 * D)),
        ],
        out_specs=pl.BlockSpec((NBLK, 2, _TB, 96), lambda g: (0, 0, g, 0)),
        out_shape=jax.ShapeDtypeStruct((NBLK, 2, BINS, 96), jnp.bfloat16),
    )(bond_W, awp, WCG, bcg)


_NB = 512


def _ntrans_body(x_ref, m_ref, t_ref):
    z = jnp.concatenate([x_ref[0], x_ref[1]], axis=-1)
    for t in range(4):
        t_ref[t] = jnp.dot(
            z, m_ref[t], preferred_element_type=jnp.float32
        ).astype(jnp.bfloat16)


def _node_transform(X, M):
    return pl.pallas_call(
        _ntrans_body,
        grid=(NPAD // _NB,),
        in_specs=[
            pl.BlockSpec((2, _NB, H), lambda n: (0, n, 0)),
            pl.BlockSpec((4, D, D), lambda n: (0, 0, 0)),
        ],
        out_specs=pl.BlockSpec((4, _NB, D), lambda n: (0, n, 0)),
        out_shape=jax.ShapeDtypeStruct((4, NPAD, D), jnp.bfloat16),
    )(X, M)


def _make_edge_body(blk):
    def body(t_hbm, uidx_hbm, vidx_hbm, dst_hbm, qidx_hbm, cw_hbm,
             xin_hbm, xout_hbm,
             acc, ubuf, vbuf, cwbuf, msgb, uix, vix, cix, dix,
             semL0, semL1, semI0, semI1, semS0, semS1, semD0, semD1):
        c = lax.axis_index("c")
        s = lax.axis_index("s")
        rbase = s * ROWS_PER_TILE
        ebase = s * EPT
        semL = (semL0, semL1)
        semI = (semI0, semI1)
        semS = (semS0, semS1)
        semD = (semD0, semD1)

        @pl.loop(0, NXCH)
        def _init(j):
            off = rbase + j * CH
            pltpu.sync_copy(xin_hbm.at[c, pl.ds(off, CH), :], msgb.at[0])
            pltpu.sync_copy(msgb.at[0], acc.at[pl.ds(off, CH), :])

        plsc.subcore_barrier()

        def eoff(ci):
            return ebase + lax.rem(ci, NCHUNK) * CH

        def issue_idx(ci, slot, sem):
            o = eoff(ci)
            pltpu.async_copy(uidx_hbm.at[c, pl.ds(o, CH)], uix.at[slot], sem)
            pltpu.async_copy(vidx_hbm.at[c, pl.ds(o, CH)], vix.at[slot], sem)
            pltpu.async_copy(qidx_hbm.at[blk, c, pl.ds(o, CH)],
                             cix.at[slot], sem)

        def wait_idx(ci, slot, sem):
            o = eoff(ci)
            pltpu.make_async_copy(
                uidx_hbm.at[c, pl.ds(o, CH)], uix.at[slot], sem).wait()
            pltpu.make_async_copy(
                vidx_hbm.at[c, pl.ds(o, CH)], vix.at[slot], sem).wait()
            pltpu.make_async_copy(
                qidx_hbm.at[blk, c, pl.ds(o, CH)], cix.at[slot], sem).wait()

        def issue_loads(b, slot, sem):
            pltpu.async_copy(cw_hbm.at[cix.at[slot]], cwbuf.at[b], sem)
            pltpu.async_copy(t_hbm.at[uix.at[slot]], ubuf.at[b], sem)
            pltpu.async_copy(t_hbm.at[vix.at[slot]], vbuf.at[b], sem)

        def wait_loads(b, slot, sem):
            pltpu.make_async_copy(cw_hbm.at[cix.at[slot]], cwbuf.at[b],
                                  sem).wait()
            pltpu.make_async_copy(t_hbm.at[uix.at[slot]], ubuf.at[b],
                                  sem).wait()
            pltpu.make_async_copy(t_hbm.at[vix.at[slot]], vbuf.at[b],
                                  sem).wait()

        pltpu.sync_copy(uidx_hbm.at[c, pl.ds(ebase, CH)], uix.at[0])
        pltpu.sync_copy(vidx_hbm.at[c, pl.ds(ebase, CH)], vix.at[0])
        pltpu.sync_copy(qidx_hbm.at[blk, c, pl.ds(ebase, CH)], cix.at[0])
        pltpu.sync_copy(uidx_hbm.at[c, pl.ds(ebase + CH, CH)], uix.at[1])
        pltpu.sync_copy(vidx_hbm.at[c, pl.ds(ebase + CH, CH)], vix.at[1])
        pltpu.sync_copy(qidx_hbm.at[blk, c, pl.ds(ebase + CH, CH)], cix.at[1])
        issue_idx(2, 2, semI[0])
        issue_idx(3, 3, semI[1])
        issue_loads(0, 0, semL[0])
        issue_loads(1, 1, semL[1])
        zeros16 = jnp.zeros((16,), jnp.float32)
        trash = jnp.full((16,), NPAD, jnp.int32)
        for b in range(2):
            @pl.loop(0, CH)
            def _z(r):
                msgb[b, r, pl.ds(0, 16)] = zeros16
                msgb[b, r, pl.ds(16, 16)] = zeros16

            for j in range(CH // 16):
                dix[b, pl.ds(j * 16, 16)] = trash
            pltpu.async_copy(msgb.at[b], acc.at[dix.at[b]], semS[b], add=True)

        @pl.loop(0, NCHUNK // 4)
        def _quad(g):
            for u in range(4):
                b = u % 2
                ci = g * 4 + u
                pltpu.make_async_copy(msgb.at[b], acc.at[dix.at[b]],
                                      semS[b]).wait()
                pltpu.async_copy(dst_hbm.at[pl.ds(eoff(ci), CH)],
                                 dix.at[b], semD[b])
                wait_loads(b, u, semL[b])

                @plsc.parallel_loop(0, CH, unroll=4)
                def _edge(ii):
                    wi = plsc.bitcast(cwbuf[b, ii, pl.ds(2 * H, H)], jnp.int32)
                    for k in range(2):
                        sl32 = pl.ds(k * H, H)
                        ui = plsc.bitcast(ubuf[b, ii, sl32], jnp.int32)
                        vi = plsc.bitcast(vbuf[b, ii, sl32], jnp.int32)
                        ci_ = plsc.bitcast(cwbuf[b, ii, sl32], jnp.int32)
                        cp = _lo(ui) + _lo(vi) + _lo(ci_)
                        gp = _hi(ui) + _hi(vi) + _hi(ci_)
                        w = _lo(wi) if k == 0 else _hi(wi)
                        den = (1.0 + jnp.exp(-cp)) * (1.0 + jnp.exp(-gp))
                        msgb[b, ii, pl.ds(k * 16, 16)] = cp * w / den

                pltpu.make_async_copy(dst_hbm.at[pl.ds(eoff(ci), CH)],
                                      dix.at[b], semD[b]).wait()
                pltpu.async_copy(msgb.at[b], acc.at[dix.at[b]],
                                 semS[b], add=True)
                wait_idx(ci + 2, (u + 2) % 4, semI[b])
                issue_loads(b, (u + 2) % 4, semL[b])
                issue_idx(ci + 4, u, semI[b])

        for u in range(2):
            b = u % 2
            pltpu.make_async_copy(msgb.at[b], acc.at[dix.at[b]],
                                  semS[b]).wait()
            wait_loads(b, (u + 2) % 4, semL[b])
            wait_idx(NCHUNK + u + 2, u, semI[b])

        plsc.subcore_barrier()

        @pl.loop(0, NXCH)
        def _out(j):
            off = rbase + j * CH
            pltpu.sync_copy(acc.at[pl.ds(off, CH), :], msgb.at[0])
            pltpu.sync_copy(msgb.at[0], xout_hbm.at[c, pl.ds(off, CH), :])

    return body


def _edge_kernel(blk):
    return pl.kernel(
        _make_edge_body(blk),
        out_type=jax.ShapeDtypeStruct((2, NPAD, H), jnp.float32),
        mesh=_MESH,
        scratch_types=[
            pltpu.VMEM_SHARED((NPAD + 8, H), jnp.float32),
            pltpu.VMEM((2, CH, D), jnp.bfloat16),
            pltpu.VMEM((2, CH, D), jnp.bfloat16),
            pltpu.VMEM((2, CH, 96), jnp.bfloat16),
            pltpu.VMEM((2, CH, H), jnp.float32),
            pltpu.VMEM((4, CH), jnp.int32),
            pltpu.VMEM((4, CH), jnp.int32),
            pltpu.VMEM((4, CH), jnp.int32),
            pltpu.VMEM((2, CH), jnp.int32),
            pltpu.SemaphoreType.DMA,
            pltpu.SemaphoreType.DMA,
            pltpu.SemaphoreType.DMA,
            pltpu.SemaphoreType.DMA,
            pltpu.SemaphoreType.DMA,
            pltpu.SemaphoreType.DMA,
            pltpu.SemaphoreType.DMA,
            pltpu.SemaphoreType.DMA,
        ],
        compiler_params=_SC_PARAMS,
    )


_RB = 1000


def _readout_body(x_ref, r1w, r1b, r2w, r2b, r3w, r3b, sw, sb,
                  site_ref, en_ref):
    z = jnp.concatenate([x_ref[0], x_ref[1]], axis=-1)
    site_ref[...] = jnp.dot(z, sw[...],
                            preferred_element_type=jnp.float32) + sb[...]
    h = jnp.dot(z, r1w[...], preferred_element_type=jnp.float32) + r1b[...]
    h = h / (1.0 + jnp.exp(-h))
    h = jnp.dot(h, r2w[...], preferred_element_type=jnp.float32) + r2b[...]
    h = h / (1.0 + jnp.exp(-h))
    pn = jnp.dot(h, r3w[...], preferred_element_type=jnp.float32) + r3b[...]
    en = jnp.sum(pn)

    @pl.when(pl.program_id(0) == 0)
    def _():
        en_ref[...] = jnp.zeros((1, 1), jnp.float32)

    en_ref[...] += jnp.reshape(en, (1, 1))


def _readout(X, R1W, R1b, R2W, R2b, R3W, R3b, siteW, siteb):
    full = lambda shape: pl.BlockSpec(shape, lambda n: tuple(0 for _ in shape))
    return pl.pallas_call(
        _readout_body,
        grid=(N // _RB,),
        in_specs=[
            pl.BlockSpec((2, _RB, H), lambda n: (0, n, 0)),
            full((D, D)), full((1, D)),
            full((D, D)), full((1, D)),
            full((D, 1)), full((1, 1)),
            full((D, 1)), full((1, 1)),
        ],
        out_specs=[
            pl.BlockSpec((_RB, 1), lambda n: (n, 0)),
            pl.BlockSpec((1, 1), lambda n: (0, 0)),
        ],
        out_shape=[
            jax.ShapeDtypeStruct((N, 1), jnp.float32),
            jax.ShapeDtypeStruct((1, 1), jnp.float32),
        ],
    )(X, R1W, R1b, R2W, R2b, R3W, R3b, siteW, siteb)


def kernel(node_types, edge_index, bond_dist, atom_emb, bond_W, abw_W,
           Wc, bc, Wg, bg, R1W, R1b, R2W, R2b, R3W, R3b, siteW, siteb):
    f32 = jnp.float32
    src = edge_index[0].astype(jnp.int32)
    dst = edge_index[1].astype(jnp.int32)
    nt = node_types.astype(jnp.int32)

    src_p = jnp.pad(src, (0, EPAD - E))
    dst_p = jnp.pad(dst, (0, EPAD - E))
    bd_p = jnp.pad(bond_dist.astype(f32), (0, EPAD - E),
                   constant_values=2.0 * CUTOFF)
    nt_p = jnp.pad(nt, (0, NPAD - N))

    uidx = jnp.stack([src_p, src_p + NPAD])
    vidx = jnp.stack([dst_p + 2 * NPAD, dst_p + 3 * NPAD])
    nidx = jnp.stack([nt_p, nt_p + NELEM])

    embS = jnp.concatenate([atom_emb[:, :H], atom_emb[:, H:]], axis=0)
    embS = embS.astype(f32)
    wcc = Wc[:, 2 * D:, :].reshape(NBLK, D, 2, 2, 16)
    wgc = Wg[:, 2 * D:, :].reshape(NBLK, D, 2, 2, 16)
    WCG = jnp.stack([wcc, wgc], axis=-1).reshape(NBLK, D, 2 * D).astype(f32)
    bcr = bc.reshape(NBLK, 2, 2, 16)
    bgr = bg.reshape(NBLK, 2, 2, 16)
    bcg = jnp.stack([bcr, bgr], axis=-1).reshape(NBLK, 1, 2 * D).astype(f32)
    awp = abw_W.reshape(MAXN, 2, 2, 16).transpose(0, 1, 3, 2)
    awp = awp.reshape(MAXN, D).astype(f32)
    Ms = []
    for i in range(NBLK):
        wa_c = Wc[i, :D, :].reshape(D, 2, 2, 16)
        wb_c = Wc[i, D:2 * D, :].reshape(D, 2, 2, 16)
        wa_g = Wg[i, :D, :].reshape(D, 2, 2, 16)
        wb_g = Wg[i, D:2 * D, :].reshape(D, 2, 2, 16)
        tbls = [
            jnp.stack([wa_c[:, 0], wa_g[:, 0]], axis=-1).reshape(D, D),
            jnp.stack([wa_c[:, 1], wa_g[:, 1]], axis=-1).reshape(D, D),
            jnp.stack([wb_c[:, 0], wb_g[:, 0]], axis=-1).reshape(D, D),
            jnp.stack([wb_c[:, 1], wb_g[:, 1]], axis=-1).reshape(D, D),
        ]
        Ms.append(jnp.stack(tbls).astype(f32))

    QIDX = _qidx_kernel(bd_p.reshape(1, EPAD))
    CW = _table_kernel(bond_W.astype(f32), awp, WCG, bcg)
    CWflat = CW.reshape(NBLK * 2 * BINS, 96)

    X = _emb_kernel(embS, nidx)

    for i in range(NBLK):
        Tflat = _node_transform(X, Ms[i]).reshape(4 * NPAD, D)
        X = _edge_kernel(i)(Tflat, uidx, vidx, dst_p, QIDX, CWflat, X)

    site, en = _readout(X, R1W.astype(f32), R1b.reshape(1, D).astype(f32),
                        R2W.astype(f32), R2b.reshape(1, D).astype(f32),
                        R3W.astype(f32), R3b.reshape(1, 1).astype(f32),
                        siteW.astype(f32), siteb.reshape(1, 1).astype(f32))
    return (en.reshape(1), site)
